# baseline (speedup 1.0000x reference)
import functools

import jax
import jax.numpy as jnp
from jax import lax
from jax.experimental import pallas as pl
from jax.experimental.pallas import tpu as pltpu
from jax.experimental.pallas import tpu_sc as plsc

NC, NS = 2, 16
CHUNK = 128
F32 = jnp.float32


def _relu(x):
    return jnp.maximum(x, 0.0)


def _layernorm(y, gamma, beta, axis):
    mu = jnp.mean(y, axis=axis, keepdims=True)
    var = jnp.mean((y - mu) ** 2, axis=axis, keepdims=True)
    return (y - mu) * jax.lax.rsqrt(var + 1e-5) * gamma + beta


def _dot(a, b):
    return jnp.dot(a, b, preferred_element_type=F32)


def _dot_c00(a, b):
    return lax.dot_general(a, b, (((0,), (0,)), ((), ())),
                           preferred_element_type=F32)


def _dot_c01(a, b):
    return lax.dot_general(a, b, (((0,), (1,)), ((), ())),
                           preferred_element_type=F32)


def _full_spec(shape):
    return pl.BlockSpec(shape, lambda i: tuple(0 for _ in shape))


def _tc_prep(xn, gcat, Wr, Ws, Wge, b1e, Wgn, b1n, bn=2000):
    N, dn = xn.shape
    Lp = Wr.shape[1]
    Ln = Wgn.shape[1]
    dg = gcat.shape[1]
    grid = N // bn

    def body(x_ref, g_ref, wr_ref, ws_ref, wge_ref, b1e_ref, wgn_ref, b1n_ref,
             a_ref, b_ref, ge_ref, gn_ref):
        x = x_ref[...]
        a_ref[...] = _dot(x, wr_ref[...])
        b_ref[...] = _dot(x, ws_ref[...])
        ge_ref[...] = _dot(g_ref[...], wge_ref[...]) + b1e_ref[...]
        gn_ref[...] = _dot(g_ref[...], wgn_ref[...]) + b1n_ref[...]

    return pl.pallas_call(
        body,
        grid=(grid,),
        in_specs=[
            pl.BlockSpec((bn, dn), lambda i: (i, 0)),
            _full_spec((1, dg)),
            _full_spec((dn, Lp)),
            _full_spec((dn, Lp)),
            _full_spec((dg, Lp)),
            _full_spec((1, Lp)),
            _full_spec((dg, Ln)),
            _full_spec((1, Ln)),
        ],
        out_specs=[
            pl.BlockSpec((bn, Lp), lambda i: (i, 0)),
            pl.BlockSpec((bn, Lp), lambda i: (i, 0)),
            _full_spec((1, Lp)),
            _full_spec((1, Ln)),
        ],
        out_shape=[
            jax.ShapeDtypeStruct((N, Lp), F32),
            jax.ShapeDtypeStruct((N, Lp), F32),
            jax.ShapeDtypeStruct((1, Lp), F32),
            jax.ShapeDtypeStruct((1, Ln), F32),
        ],
    )(xn, gcat, Wr, Ws, Wge, b1e, Wgn, b1n)


def _sc_gather(A, B, rowi, coli):
    N, Lp = A.shape
    M = rowi.shape[0]
    nch = M // CHUNK
    nw = NC * NS
    kmax = -(-nch // nw)
    mesh = plsc.VectorSubcoreMesh(core_axis_name="c", subcore_axis_name="s",
                                  num_cores=NC, num_subcores=NS)

    @functools.partial(
        pl.kernel,
        out_type=jax.ShapeDtypeStruct((2, M, Lp), F32),
        mesh=mesh,
        scratch_types=[
            pltpu.VMEM((CHUNK,), jnp.int32),
            pltpu.VMEM((CHUNK,), jnp.int32),
            pltpu.VMEM((CHUNK, Lp), F32),
            pltpu.VMEM((CHUNK, Lp), F32),
            pltpu.SemaphoreType.DMA,
        ],
    )
    def k(a_hbm, b_hbm, row_hbm, col_hbm, out_hbm, rowv, colv, bufa, bufb,
          sem):
        w = lax.axis_index("s") * NC + lax.axis_index("c")

        @pl.loop(0, kmax)
        def _chunks(kk):
            ch = kk * nw + w

            @pl.when(ch < nch)
            def _():
                base = ch * CHUNK
                pltpu.sync_copy(row_hbm.at[pl.ds(base, CHUNK)], rowv)
                pltpu.sync_copy(col_hbm.at[pl.ds(base, CHUNK)], colv)
                ca = pltpu.async_copy(a_hbm.at[colv], bufa, sem)
                cb = pltpu.async_copy(b_hbm.at[rowv], bufb, sem)
                ca.wait()
                cb.wait()
                pltpu.sync_copy(bufa, out_hbm.at[0, pl.ds(base, CHUNK)])
                pltpu.sync_copy(bufb, out_hbm.at[1, pl.ds(base, CHUNK)])

    return k(A, B, rowi, coli)


def _sc_scatter_wide(eT, rowi, coli, zeros, n_nodes):
    D, M = eT.shape
    FPW = D // (NC * NS)
    nch = M // CHUNK
    eT_flat = eT.reshape(-1)
    mesh = plsc.VectorSubcoreMesh(core_axis_name="c", subcore_axis_name="s",
                                  num_cores=NC, num_subcores=NS)

    @functools.partial(
        pl.kernel,
        out_type=[jax.ShapeDtypeStruct((D * n_nodes,), F32)] * 2,
        mesh=mesh,
        compiler_params=pltpu.CompilerParams(needs_layout_passes=False),
        scratch_types=(
            [pltpu.VMEM((CHUNK,), jnp.int32)] * 2
            + [pltpu.VMEM((CHUNK,), F32)] * FPW
            + [pltpu.VMEM((n_nodes,), F32)] * (2 * FPW)
        ),
    )
    def k(e_hbm, row_hbm, col_hbm, z_hbm, s_hbm, r_hbm, rowv, colv, *bufs):
        dbuf = bufs[:FPW]
        accs = bufs[FPW:2 * FPW]
        accr = bufs[2 * FPW:3 * FPW]
        w = lax.axis_index("s") * NC + lax.axis_index("c")
        f0 = w * FPW
        for j in range(FPW):
            pltpu.sync_copy(z_hbm.at[0], accs[j])
            pltpu.sync_copy(z_hbm.at[0], accr[j])

        @pl.loop(0, nch)
        def _chunks(ch):
            base = ch * CHUNK
            pltpu.sync_copy(row_hbm.at[pl.ds(base, CHUNK)], rowv)
            pltpu.sync_copy(col_hbm.at[pl.ds(base, CHUNK)], colv)
            for j in range(FPW):
                pltpu.sync_copy(e_hbm.at[pl.ds((f0 + j) * M + base, CHUNK)],
                                dbuf[j])
            for v in range(CHUNK // 16):
                sl = pl.ds(v * 16, 16)
                r16 = rowv[sl]
                c16 = colv[sl]
                for j in range(FPW):
                    x16 = dbuf[j][sl]
                    plsc.addupdate_scatter(accs[j], [r16], x16)
                    plsc.addupdate_scatter(accr[j], [c16], x16)

        for j in range(FPW):
            pltpu.sync_copy(accs[j],
                            s_hbm.at[pl.ds((f0 + j) * n_nodes, n_nodes)])
            pltpu.sync_copy(accr[j],
                            r_hbm.at[pl.ds((f0 + j) * n_nodes, n_nodes)])

    s_flat, r_flat = k(eT_flat, rowi, coli, zeros[:FPW])
    return s_flat.reshape(D, n_nodes), r_flat.reshape(D, n_nodes)


def _sc_scatter_narrow(eT, rowi, coli, zeros, n_nodes):
    D, M = eT.shape
    nch = M // CHUNK
    eT_flat = eT.reshape(-1)
    mesh = plsc.VectorSubcoreMesh(core_axis_name="c", subcore_axis_name="s",
                                  num_cores=NC, num_subcores=NS)

    @functools.partial(
        pl.kernel,
        out_type=[jax.ShapeDtypeStruct((D * n_nodes,), F32)] * 2,
        mesh=mesh,
        compiler_params=pltpu.CompilerParams(needs_layout_passes=False),
        scratch_types=[
            pltpu.VMEM((CHUNK,), jnp.int32),
            pltpu.VMEM((CHUNK,), F32),
            pltpu.VMEM((n_nodes,), F32),
        ],
    )
    def k(e_hbm, row_hbm, col_hbm, z_hbm, s_hbm, r_hbm, idxv, dbuf, acc):
        c = lax.axis_index("c")
        s = lax.axis_index("s")
        pltpu.sync_copy(z_hbm.at[0], acc)

        @pl.loop(0, nch)
        def _chunks(ch):
            base = ch * CHUNK

            @pl.when(c == 0)
            def _():
                pltpu.sync_copy(row_hbm.at[pl.ds(base, CHUNK)], idxv)

            @pl.when(c == 1)
            def _():
                pltpu.sync_copy(col_hbm.at[pl.ds(base, CHUNK)], idxv)

            pltpu.sync_copy(e_hbm.at[pl.ds(s * M + base, CHUNK)], dbuf)
            for v in range(CHUNK // 16):
                sl = pl.ds(v * 16, 16)
                plsc.addupdate_scatter(acc, [idxv[sl]], dbuf[sl])

        @pl.when(c == 0)
        def _():
            pltpu.sync_copy(acc, s_hbm.at[pl.ds(s * n_nodes, n_nodes)])

        @pl.when(c == 1)
        def _():
            pltpu.sync_copy(acc, r_hbm.at[pl.ds(s * n_nodes, n_nodes)])

    s_flat, r_flat = k(eT_flat, rowi, coli, zeros[:1])
    return s_flat.reshape(D, n_nodes), r_flat.reshape(D, n_nodes)


def _tc_edge(G, e_pieces, W1e_pieces, gvec, W2p, b2T, gammaT, betaT, act,
             norm, narrow, bm=3200):
    _, M, Lp = G.shape
    dout = W2p.shape[1]
    grid = M // bm
    ne = len(e_pieces)

    def body(*refs):
        g_ref = refs[0]
        e_refs = refs[1:1 + ne]
        w1_refs = refs[1 + ne:1 + 2 * ne]
        gvec_ref, w2_ref, b2t_ref = refs[1 + 2 * ne:4 + 2 * ne]
        if narrow:
            outT_ref, outN_ref, esum_ref = refs[-3:]
        else:
            outT_ref, esum_ref = refs[-2:]
        g = g_ref[...]
        h = g[0] + g[1] + gvec_ref[...]
        for e_ref, w1_ref in zip(e_refs, w1_refs):
            e = e_ref[...]
            if e.shape[0] == bm:
                h = h + _dot(e, w1_ref[...])
            else:
                h = h + _dot_c00(e, w1_ref[...])
        h = _relu(h)
        yT = _dot_c01(w2_ref[...], h) + b2t_ref[...]
        if act:
            yT = _relu(yT)
        if norm:
            yT = _layernorm(yT, refs[4 + 2 * ne][...], refs[5 + 2 * ne][...],
                            axis=0)
        outT_ref[...] = yT
        if narrow:
            outN_ref[...] = yT.T
        i = pl.program_id(0)

        @pl.when(i == 0)
        def _():
            esum_ref[...] = jnp.zeros_like(esum_ref)

        esum_ref[...] += jnp.sum(yT, axis=1, keepdims=True)

    in_specs = [pl.BlockSpec((2, bm, Lp), lambda i: (0, i, 0))]
    args = [G]
    for e in e_pieces:
        if e.shape[0] == M:
            in_specs.append(pl.BlockSpec((bm, e.shape[1]), lambda i: (i, 0)))
        else:
            in_specs.append(pl.BlockSpec((e.shape[0], bm), lambda i: (0, i)))
        args.append(e)
    for wp in W1e_pieces:
        in_specs.append(_full_spec(wp.shape))
        args.append(wp)
    for a in (gvec, W2p, b2T):
        in_specs.append(_full_spec(a.shape))
        args.append(a)
    if norm:
        for a in (gammaT, betaT):
            in_specs.append(_full_spec(a.shape))
            args.append(a)

    out_specs = [pl.BlockSpec((dout, bm), lambda i: (0, i))]
    out_shape = [jax.ShapeDtypeStruct((dout, M), F32)]
    if narrow:
        out_specs.append(pl.BlockSpec((bm, dout), lambda i: (i, 0)))
        out_shape.append(jax.ShapeDtypeStruct((M, dout), F32))
    out_specs.append(_full_spec((dout, 1)))
    out_shape.append(jax.ShapeDtypeStruct((dout, 1), F32))

    return pl.pallas_call(
        body,
        grid=(grid,),
        in_specs=in_specs,
        out_specs=out_specs,
        out_shape=out_shape,
    )(*args)


def _tc_node(xn, sendT, recvT, gvec_n, Wx, Wrecv, Wsend, W2, b2, gamma,
             beta, act, norm):
    N, dn = xn.shape
    dout = W2.shape[1]

    def body(*refs):
        (x_ref, st_ref, rt_ref, gvec_ref, wx_ref, wr_ref, ws_ref, w2_ref,
         b2_ref) = refs[:9]
        out_ref, nsum_ref = refs[-2:]
        h = (_dot(x_ref[...], wx_ref[...]) + _dot_c00(rt_ref[...], wr_ref[...])
             + _dot_c00(st_ref[...], ws_ref[...]) + gvec_ref[...])
        h = _relu(h)
        y = _dot(h, w2_ref[...]) + b2_ref[...]
        if act:
            y = _relu(y)
        if norm:
            y = _layernorm(y, refs[9][...], refs[10][...], axis=-1)
        out_ref[...] = y
        nsum_ref[...] = jnp.sum(y, axis=0, keepdims=True)

    args = [xn, sendT, recvT, gvec_n, Wx, Wrecv, Wsend, W2, b2]
    if norm:
        args += [gamma, beta]

    return pl.pallas_call(
        body,
        out_shape=[
            jax.ShapeDtypeStruct((N, dout), F32),
            jax.ShapeDtypeStruct((1, dout), F32),
        ],
    )(*args)


def _tc_global(nsum, esum, gcat, Wn, We, Wg, b1, W2, b2, gamma, beta, act,
               norm, n_nodes, n_edges):
    dout = W2.shape[1]

    def body(*refs):
        (ns_ref, es_ref, g_ref, wn_ref, we_ref, wg_ref, b1_ref, w2_ref,
         b2_ref) = refs[:9]
        out_ref = refs[-1]
        n2g = ns_ref[...] * (1.0 / n_nodes)
        e2g = es_ref[...] * (1.0 / n_edges)
        h = (_dot(n2g, wn_ref[...]) + _dot(e2g, we_ref[...])
             + _dot(g_ref[...], wg_ref[...]) + b1_ref[...])
        h = _relu(h)
        y = _dot(h, w2_ref[...]) + b2_ref[...]
        if act:
            y = _relu(y)
        if norm:
            y = _layernorm(y, refs[9][...], refs[10][...], axis=-1)
        out_ref[...] = y

    args = [nsum, esum, gcat, Wn, We, Wg, b1, W2, b2]
    if norm:
        args += [gamma, beta]
    return pl.pallas_call(
        body,
        out_shape=jax.ShapeDtypeStruct((1, dout), F32),
    )(*args)


def _pad_cols(w, Lp):
    pad = Lp - w.shape[-1]
    return jnp.pad(w, [(0, 0)] * (w.ndim - 1) + [(0, pad)]) if pad else w


def _run_block(p, e_pieces, n_pieces, g_pieces, rowi, coli, zeros, act,
               norm, narrow):
    xn = n_pieces[0] if len(n_pieces) == 1 else jnp.concatenate(n_pieces, 1)
    gcat = g_pieces[0] if len(g_pieces) == 1 else jnp.concatenate(g_pieces, 1)
    N, dn = xn.shape
    M = rowi.shape[0]

    def piece_dim(e):
        return e.shape[1] if e.shape[0] == M else e.shape[0]

    de = sum(piece_dim(e) for e in e_pieces)
    dg = gcat.shape[1]

    pe, pn, pg = p["edge"], p["node"], p["global"]
    L = pe["W1"].shape[1]
    Lp = ((L + 127) // 128) * 128

    W1 = pe["W1"]
    Wr = _pad_cols(W1[:dn], Lp)
    Ws = _pad_cols(W1[dn:2 * dn], Lp)
    We_full = W1[2 * dn:2 * dn + de]
    Wg_e = _pad_cols(W1[2 * dn + de:], Lp)
    b1e = _pad_cols(pe["b1"][None, :], Lp)
    W2e = jnp.pad(pe["W2"], ((0, Lp - L), (0, 0))) if Lp != L else pe["W2"]

    W1n = pn["W1"]
    dout_e = pe["W2"].shape[1]
    Wx = W1n[:dn]
    Wg_n = W1n[dn:dn + dg]
    Wrecv = W1n[dn + dg:dn + dg + dout_e]
    Wsend = W1n[dn + dg + dout_e:]
    b1n = pn["b1"][None, :]

    A, B, gvec_e, gvec_n = _tc_prep(xn, gcat, Wr, Ws, Wg_e, b1e, Wg_n, b1n)
    G = _sc_gather(A, B, rowi, coli)

    W1e_pieces = []
    off = 0
    for e in e_pieces:
        d = piece_dim(e)
        W1e_pieces.append(_pad_cols(We_full[off:off + d], Lp))
        off += d

    gammaT = pe["gamma"][:, None] if norm else None
    betaT = pe["beta"][:, None] if norm else None
    eout = _tc_edge(G, e_pieces, W1e_pieces, gvec_e, W2e,
                    pe["b2"][:, None], gammaT, betaT, act, norm, narrow)
    if narrow:
        eT, e_nat, esumT = eout
    else:
        eT, esumT = eout
        e_nat = None
    esum = esumT.reshape(1, dout_e)

    if narrow:
        sendT, recvT = _sc_scatter_narrow(eT, rowi, coli, zeros, N)
    else:
        sendT, recvT = _sc_scatter_wide(eT, rowi, coli, zeros, N)

    gamma_n = pn["gamma"][None, :] if norm else None
    beta_n = pn["beta"][None, :] if norm else None
    n_new, nsum = _tc_node(xn, sendT, recvT, gvec_n, Wx, Wrecv, Wsend,
                           pn["W2"], pn["b2"][None, :], gamma_n, beta_n,
                           act, norm)

    W1g = pg["W1"]
    dout_n = pn["W2"].shape[1]
    Wgn2g = W1g[:dout_n]
    Wge2g = W1g[dout_n:dout_n + dout_e]
    Wgg = W1g[dout_n + dout_e:]
    gamma_g = pg["gamma"][None, :] if norm else None
    beta_g = pg["beta"][None, :] if norm else None
    g_new = _tc_global(nsum, esum, gcat, Wgn2g, Wge2g, Wgg,
                       pg["b1"][None, :], pg["W2"], pg["b2"][None, :],
                       gamma_g, beta_g, act, norm, N, M)
    e_ret = e_nat if narrow else eT
    return e_ret, n_new, g_new


def kernel(edge_attr, edge_index, x, u, batch, params):
    rowi = edge_index[0]
    coli = edge_index[1]
    n_nodes = x.shape[0]
    zeros = jnp.zeros((4, n_nodes), F32)
    eT, n, g = _run_block(params["encoder"], [edge_attr], [x], [u], rowi,
                          coli, zeros, True, True, False)
    eT0, n0, g0 = eT, n, g
    out = None
    for _ in range(3):
        eTp, np_, gp = _run_block(params["processor"], [eT0, eT], [n0, n],
                                  [g0, g], rowi, coli, zeros, True, True,
                                  False)
        eTd, nd, gd = _run_block(params["decoder"], [eTp], [np_], [gp],
                                 rowi, coli, zeros, True, True, False)
        out = _run_block(params["output"], [eTd], [nd], [gd], rowi, coli,
                         zeros, False, False, True)
        eT, n, g = eTp, np_, gp
    return out


# speedup vs baseline: 3.9148x; 3.9148x over previous
import functools

import jax
import jax.numpy as jnp
from jax import lax
from jax.experimental import pallas as pl
from jax.experimental.pallas import tpu as pltpu
from jax.experimental.pallas import tpu_sc as plsc

NC, NS = 2, 16
CHUNK = 128
F32 = jnp.float32


def _relu(x):
    return jnp.maximum(x, 0.0)


def _layernorm(y, gamma, beta, axis):
    mu = jnp.mean(y, axis=axis, keepdims=True)
    var = jnp.mean((y - mu) ** 2, axis=axis, keepdims=True)
    return (y - mu) * jax.lax.rsqrt(var + 1e-5) * gamma + beta


def _dot(a, b):
    return jnp.dot(a, b, preferred_element_type=F32)


def _dot_c00(a, b):
    return lax.dot_general(a, b, (((0,), (0,)), ((), ())),
                           preferred_element_type=F32)


def _dot_c01(a, b):
    return lax.dot_general(a, b, (((0,), (1,)), ((), ())),
                           preferred_element_type=F32)


def _full_spec(shape):
    return pl.BlockSpec(shape, lambda i: tuple(0 for _ in shape))


def _tc_prep(xn, gcat, Wr, Ws, Wge, b1e, Wgn, b1n, bn=2000):
    N, dn = xn.shape
    Lp = Wr.shape[1]
    Ln = Wgn.shape[1]
    dg = gcat.shape[1]
    grid = N // bn

    def body(x_ref, g_ref, wr_ref, ws_ref, wge_ref, b1e_ref, wgn_ref, b1n_ref,
             a_ref, b_ref, ge_ref, gn_ref):
        x = x_ref[...]
        a_ref[...] = _dot(x, wr_ref[...])
        b_ref[...] = _dot(x, ws_ref[...])
        ge_ref[...] = _dot(g_ref[...], wge_ref[...]) + b1e_ref[...]
        gn_ref[...] = _dot(g_ref[...], wgn_ref[...]) + b1n_ref[...]

    return pl.pallas_call(
        body,
        grid=(grid,),
        in_specs=[
            pl.BlockSpec((bn, dn), lambda i: (i, 0)),
            _full_spec((1, dg)),
            _full_spec((dn, Lp)),
            _full_spec((dn, Lp)),
            _full_spec((dg, Lp)),
            _full_spec((1, Lp)),
            _full_spec((dg, Ln)),
            _full_spec((1, Ln)),
        ],
        out_specs=[
            pl.BlockSpec((bn, Lp), lambda i: (i, 0)),
            pl.BlockSpec((bn, Lp), lambda i: (i, 0)),
            _full_spec((1, Lp)),
            _full_spec((1, Ln)),
        ],
        out_shape=[
            jax.ShapeDtypeStruct((N, Lp), F32),
            jax.ShapeDtypeStruct((N, Lp), F32),
            jax.ShapeDtypeStruct((1, Lp), F32),
            jax.ShapeDtypeStruct((1, Ln), F32),
        ],
    )(xn, gcat, Wr, Ws, Wge, b1e, Wgn, b1n)


def _sc_gather(A, B, rowi, coli):
    N, Lp = A.shape
    M = rowi.shape[0]
    nch = M // CHUNK
    nw = NC * NS
    kmax = -(-nch // nw)
    mesh = plsc.VectorSubcoreMesh(core_axis_name="c", subcore_axis_name="s",
                                  num_cores=NC, num_subcores=NS)

    @functools.partial(
        pl.kernel,
        out_type=jax.ShapeDtypeStruct((2, M, Lp), F32),
        mesh=mesh,
        scratch_types=[
            pltpu.VMEM((CHUNK,), jnp.int32),
            pltpu.VMEM((CHUNK,), jnp.int32),
            pltpu.VMEM((CHUNK, Lp), F32),
            pltpu.VMEM((CHUNK, Lp), F32),
            pltpu.SemaphoreType.DMA,
        ],
    )
    def k(a_hbm, b_hbm, row_hbm, col_hbm, out_hbm, rowv, colv, bufa, bufb,
          sem):
        w = lax.axis_index("s") * NC + lax.axis_index("c")

        @pl.loop(0, kmax)
        def _chunks(kk):
            ch = kk * nw + w

            @pl.when(ch < nch)
            def _():
                base = ch * CHUNK
                pltpu.sync_copy(row_hbm.at[pl.ds(base, CHUNK)], rowv)
                pltpu.sync_copy(col_hbm.at[pl.ds(base, CHUNK)], colv)
                ca = pltpu.async_copy(a_hbm.at[colv], bufa, sem)
                cb = pltpu.async_copy(b_hbm.at[rowv], bufb, sem)
                ca.wait()
                cb.wait()
                pltpu.sync_copy(bufa, out_hbm.at[0, pl.ds(base, CHUNK)])
                pltpu.sync_copy(bufb, out_hbm.at[1, pl.ds(base, CHUNK)])

    return k(A, B, rowi, coli)


def _sc_scatter_wide(eT, rowi, coli, zeros, n_nodes):
    D, M = eT.shape
    FPW = D // (NC * NS)
    CH = 2000
    nch = M // CH
    eT_flat = eT.reshape(-1)
    mesh = plsc.VectorSubcoreMesh(core_axis_name="c", subcore_axis_name="s",
                                  num_cores=NC, num_subcores=NS)

    @functools.partial(
        pl.kernel,
        out_type=[jax.ShapeDtypeStruct((D * n_nodes,), F32)] * 2,
        mesh=mesh,
        compiler_params=pltpu.CompilerParams(needs_layout_passes=False),
        scratch_types=(
            [pltpu.VMEM((CH,), jnp.int32)] * 2
            + [pltpu.VMEM((CH,), F32)] * FPW
            + [pltpu.VMEM((n_nodes,), F32)] * (2 * FPW)
        ),
    )
    def k(e_hbm, row_hbm, col_hbm, z_hbm, s_hbm, r_hbm, rowv, colv, *bufs):
        dbuf = bufs[:FPW]
        accs = bufs[FPW:2 * FPW]
        accr = bufs[2 * FPW:3 * FPW]
        w = lax.axis_index("s") * NC + lax.axis_index("c")
        f0 = w * FPW
        for j in range(FPW):
            pltpu.sync_copy(z_hbm.at[0], accs[j])
            pltpu.sync_copy(z_hbm.at[0], accr[j])

        @pl.loop(0, nch)
        def _chunks(ch):
            base = ch * CH
            pltpu.sync_copy(row_hbm.at[pl.ds(base, CH)], rowv)
            pltpu.sync_copy(col_hbm.at[pl.ds(base, CH)], colv)
            for j in range(FPW):
                pltpu.sync_copy(e_hbm.at[pl.ds((f0 + j) * M + base, CH)],
                                dbuf[j])
            for v in range(CH // 16):
                sl = pl.ds(v * 16, 16)
                r16 = rowv[sl]
                c16 = colv[sl]
                for j in range(FPW):
                    x16 = dbuf[j][sl]
                    plsc.addupdate_scatter(accs[j], [r16], x16)
                    plsc.addupdate_scatter(accr[j], [c16], x16)

        for j in range(FPW):
            pltpu.sync_copy(accs[j],
                            s_hbm.at[pl.ds((f0 + j) * n_nodes, n_nodes)])
            pltpu.sync_copy(accr[j],
                            r_hbm.at[pl.ds((f0 + j) * n_nodes, n_nodes)])

    s_flat, r_flat = k(eT_flat, rowi, coli, zeros[:FPW])
    return s_flat.reshape(D, n_nodes), r_flat.reshape(D, n_nodes)


def _sc_scatter_narrow(eT, rowi, coli, zeros, n_nodes):
    D, M = eT.shape
    CH = 2000
    nch = M // CH
    eT_flat = eT.reshape(-1)
    mesh = plsc.VectorSubcoreMesh(core_axis_name="c", subcore_axis_name="s",
                                  num_cores=NC, num_subcores=NS)

    @functools.partial(
        pl.kernel,
        out_type=[jax.ShapeDtypeStruct((D * n_nodes,), F32)] * 2,
        mesh=mesh,
        compiler_params=pltpu.CompilerParams(needs_layout_passes=False),
        scratch_types=[
            pltpu.VMEM((CH,), jnp.int32),
            pltpu.VMEM((CH,), F32),
            pltpu.VMEM((n_nodes,), F32),
        ],
    )
    def k(e_hbm, row_hbm, col_hbm, z_hbm, s_hbm, r_hbm, idxv, dbuf, acc):
        c = lax.axis_index("c")
        s = lax.axis_index("s")
        pltpu.sync_copy(z_hbm.at[0], acc)

        @pl.loop(0, nch)
        def _chunks(ch):
            base = ch * CH

            @pl.when(c == 0)
            def _():
                pltpu.sync_copy(row_hbm.at[pl.ds(base, CH)], idxv)

            @pl.when(c == 1)
            def _():
                pltpu.sync_copy(col_hbm.at[pl.ds(base, CH)], idxv)

            pltpu.sync_copy(e_hbm.at[pl.ds(s * M + base, CH)], dbuf)
            for v in range(CH // 16):
                sl = pl.ds(v * 16, 16)
                plsc.addupdate_scatter(acc, [idxv[sl]], dbuf[sl])

        @pl.when(c == 0)
        def _():
            pltpu.sync_copy(acc, s_hbm.at[pl.ds(s * n_nodes, n_nodes)])

        @pl.when(c == 1)
        def _():
            pltpu.sync_copy(acc, r_hbm.at[pl.ds(s * n_nodes, n_nodes)])

    s_flat, r_flat = k(eT_flat, rowi, coli, zeros[:1])
    return s_flat.reshape(D, n_nodes), r_flat.reshape(D, n_nodes)


def _tc_edge(G, e_pieces, W1e_pieces, gvec, W2p, b2T, gammaT, betaT, act,
             norm, narrow, bm=3200):
    _, M, Lp = G.shape
    dout = W2p.shape[1]
    grid = M // bm
    ne = len(e_pieces)

    def body(*refs):
        g_ref = refs[0]
        e_refs = refs[1:1 + ne]
        w1_refs = refs[1 + ne:1 + 2 * ne]
        gvec_ref, w2_ref, b2t_ref = refs[1 + 2 * ne:4 + 2 * ne]
        if narrow:
            outT_ref, outN_ref, esum_ref = refs[-3:]
        else:
            outT_ref, esum_ref = refs[-2:]
        g = g_ref[...]
        h = g[0] + g[1] + gvec_ref[...]
        for e_ref, w1_ref in zip(e_refs, w1_refs):
            e = e_ref[...]
            if e.shape[0] == bm:
                h = h + _dot(e, w1_ref[...])
            else:
                h = h + _dot_c00(e, w1_ref[...])
        h = _relu(h)
        yT = _dot_c01(w2_ref[...], h) + b2t_ref[...]
        if act:
            yT = _relu(yT)
        if norm:
            yT = _layernorm(yT, refs[4 + 2 * ne][...], refs[5 + 2 * ne][...],
                            axis=0)
        outT_ref[...] = yT
        if narrow:
            outN_ref[...] = yT.T
        i = pl.program_id(0)

        @pl.when(i == 0)
        def _():
            esum_ref[...] = jnp.zeros_like(esum_ref)

        esum_ref[...] += jnp.sum(yT, axis=1, keepdims=True)

    in_specs = [pl.BlockSpec((2, bm, Lp), lambda i: (0, i, 0))]
    args = [G]
    for e in e_pieces:
        if e.shape[0] == M:
            in_specs.append(pl.BlockSpec((bm, e.shape[1]), lambda i: (i, 0)))
        else:
            in_specs.append(pl.BlockSpec((e.shape[0], bm), lambda i: (0, i)))
        args.append(e)
    for wp in W1e_pieces:
        in_specs.append(_full_spec(wp.shape))
        args.append(wp)
    for a in (gvec, W2p, b2T):
        in_specs.append(_full_spec(a.shape))
        args.append(a)
    if norm:
        for a in (gammaT, betaT):
            in_specs.append(_full_spec(a.shape))
            args.append(a)

    out_specs = [pl.BlockSpec((dout, bm), lambda i: (0, i))]
    out_shape = [jax.ShapeDtypeStruct((dout, M), F32)]
    if narrow:
        out_specs.append(pl.BlockSpec((bm, dout), lambda i: (i, 0)))
        out_shape.append(jax.ShapeDtypeStruct((M, dout), F32))
    out_specs.append(_full_spec((dout, 1)))
    out_shape.append(jax.ShapeDtypeStruct((dout, 1), F32))

    return pl.pallas_call(
        body,
        grid=(grid,),
        in_specs=in_specs,
        out_specs=out_specs,
        out_shape=out_shape,
    )(*args)


def _tc_node(xn, sendT, recvT, gvec_n, Wx, Wrecv, Wsend, W2, b2, gamma,
             beta, act, norm):
    N, dn = xn.shape
    dout = W2.shape[1]

    def body(*refs):
        (x_ref, st_ref, rt_ref, gvec_ref, wx_ref, wr_ref, ws_ref, w2_ref,
         b2_ref) = refs[:9]
        out_ref, nsum_ref = refs[-2:]
        h = (_dot(x_ref[...], wx_ref[...]) + _dot_c00(rt_ref[...], wr_ref[...])
             + _dot_c00(st_ref[...], ws_ref[...]) + gvec_ref[...])
        h = _relu(h)
        y = _dot(h, w2_ref[...]) + b2_ref[...]
        if act:
            y = _relu(y)
        if norm:
            y = _layernorm(y, refs[9][...], refs[10][...], axis=-1)
        out_ref[...] = y
        nsum_ref[...] = jnp.sum(y, axis=0, keepdims=True)

    args = [xn, sendT, recvT, gvec_n, Wx, Wrecv, Wsend, W2, b2]
    if norm:
        args += [gamma, beta]

    return pl.pallas_call(
        body,
        out_shape=[
            jax.ShapeDtypeStruct((N, dout), F32),
            jax.ShapeDtypeStruct((1, dout), F32),
        ],
    )(*args)


def _tc_global(nsum, esum, gcat, Wn, We, Wg, b1, W2, b2, gamma, beta, act,
               norm, n_nodes, n_edges):
    dout = W2.shape[1]

    def body(*refs):
        (ns_ref, es_ref, g_ref, wn_ref, we_ref, wg_ref, b1_ref, w2_ref,
         b2_ref) = refs[:9]
        out_ref = refs[-1]
        n2g = ns_ref[...] * (1.0 / n_nodes)
        e2g = es_ref[...] * (1.0 / n_edges)
        h = (_dot(n2g, wn_ref[...]) + _dot(e2g, we_ref[...])
             + _dot(g_ref[...], wg_ref[...]) + b1_ref[...])
        h = _relu(h)
        y = _dot(h, w2_ref[...]) + b2_ref[...]
        if act:
            y = _relu(y)
        if norm:
            y = _layernorm(y, refs[9][...], refs[10][...], axis=-1)
        out_ref[...] = y

    args = [nsum, esum, gcat, Wn, We, Wg, b1, W2, b2]
    if norm:
        args += [gamma, beta]
    return pl.pallas_call(
        body,
        out_shape=jax.ShapeDtypeStruct((1, dout), F32),
    )(*args)


def _pad_cols(w, Lp):
    pad = Lp - w.shape[-1]
    return jnp.pad(w, [(0, 0)] * (w.ndim - 1) + [(0, pad)]) if pad else w


def _run_block(p, e_pieces, n_pieces, g_pieces, rowi, coli, zeros, act,
               norm, narrow):
    xn = n_pieces[0] if len(n_pieces) == 1 else jnp.concatenate(n_pieces, 1)
    gcat = g_pieces[0] if len(g_pieces) == 1 else jnp.concatenate(g_pieces, 1)
    N, dn = xn.shape
    M = rowi.shape[0]

    def piece_dim(e):
        return e.shape[1] if e.shape[0] == M else e.shape[0]

    de = sum(piece_dim(e) for e in e_pieces)
    dg = gcat.shape[1]

    pe, pn, pg = p["edge"], p["node"], p["global"]
    L = pe["W1"].shape[1]
    Lp = ((L + 127) // 128) * 128

    W1 = pe["W1"]
    Wr = _pad_cols(W1[:dn], Lp)
    Ws = _pad_cols(W1[dn:2 * dn], Lp)
    We_full = W1[2 * dn:2 * dn + de]
    Wg_e = _pad_cols(W1[2 * dn + de:], Lp)
    b1e = _pad_cols(pe["b1"][None, :], Lp)
    W2e = jnp.pad(pe["W2"], ((0, Lp - L), (0, 0))) if Lp != L else pe["W2"]

    W1n = pn["W1"]
    dout_e = pe["W2"].shape[1]
    Wx = W1n[:dn]
    Wg_n = W1n[dn:dn + dg]
    Wrecv = W1n[dn + dg:dn + dg + dout_e]
    Wsend = W1n[dn + dg + dout_e:]
    b1n = pn["b1"][None, :]

    A, B, gvec_e, gvec_n = _tc_prep(xn, gcat, Wr, Ws, Wg_e, b1e, Wg_n, b1n)
    G = _sc_gather(A, B, rowi, coli)

    W1e_pieces = []
    off = 0
    for e in e_pieces:
        d = piece_dim(e)
        W1e_pieces.append(_pad_cols(We_full[off:off + d], Lp))
        off += d

    gammaT = pe["gamma"][:, None] if norm else None
    betaT = pe["beta"][:, None] if norm else None
    eout = _tc_edge(G, e_pieces, W1e_pieces, gvec_e, W2e,
                    pe["b2"][:, None], gammaT, betaT, act, norm, narrow)
    if narrow:
        eT, e_nat, esumT = eout
    else:
        eT, esumT = eout
        e_nat = None
    esum = esumT.reshape(1, dout_e)

    if narrow:
        sendT, recvT = _sc_scatter_narrow(eT, rowi, coli, zeros, N)
    else:
        sendT, recvT = _sc_scatter_wide(eT, rowi, coli, zeros, N)

    gamma_n = pn["gamma"][None, :] if norm else None
    beta_n = pn["beta"][None, :] if norm else None
    n_new, nsum = _tc_node(xn, sendT, recvT, gvec_n, Wx, Wrecv, Wsend,
                           pn["W2"], pn["b2"][None, :], gamma_n, beta_n,
                           act, norm)

    W1g = pg["W1"]
    dout_n = pn["W2"].shape[1]
    Wgn2g = W1g[:dout_n]
    Wge2g = W1g[dout_n:dout_n + dout_e]
    Wgg = W1g[dout_n + dout_e:]
    gamma_g = pg["gamma"][None, :] if norm else None
    beta_g = pg["beta"][None, :] if norm else None
    g_new = _tc_global(nsum, esum, gcat, Wgn2g, Wge2g, Wgg,
                       pg["b1"][None, :], pg["W2"], pg["b2"][None, :],
                       gamma_g, beta_g, act, norm, N, M)
    e_ret = e_nat if narrow else eT
    return e_ret, n_new, g_new


def kernel(edge_attr, edge_index, x, u, batch, params):
    rowi = edge_index[0]
    coli = edge_index[1]
    n_nodes = x.shape[0]
    zeros = jnp.zeros((4, n_nodes), F32)
    eT, n, g = _run_block(params["encoder"], [edge_attr], [x], [u], rowi,
                          coli, zeros, True, True, False)
    eT0, n0, g0 = eT, n, g
    out = None
    for _ in range(3):
        eTp, np_, gp = _run_block(params["processor"], [eT0, eT], [n0, n],
                                  [g0, g], rowi, coli, zeros, True, True,
                                  False)
        eTd, nd, gd = _run_block(params["decoder"], [eTp], [np_], [gp],
                                 rowi, coli, zeros, True, True, False)
        out = _run_block(params["output"], [eTd], [nd], [gd], rowi, coli,
                         zeros, False, False, True)
        eT, n, g = eTp, np_, gp
    return out


# speedup vs baseline: 4.1102x; 1.0499x over previous
import functools

import jax
import jax.numpy as jnp
from jax import lax
from jax.experimental import pallas as pl
from jax.experimental.pallas import tpu as pltpu
from jax.experimental.pallas import tpu_sc as plsc

NC, NS = 2, 16
CHUNK = 128
F32 = jnp.float32


def _relu(x):
    return jnp.maximum(x, 0.0)


def _layernorm(y, gamma, beta, axis):
    mu = jnp.mean(y, axis=axis, keepdims=True)
    var = jnp.mean((y - mu) ** 2, axis=axis, keepdims=True)
    return (y - mu) * jax.lax.rsqrt(var + 1e-5) * gamma + beta


def _dot(a, b):
    return jnp.dot(a, b, preferred_element_type=F32)


def _dot_c00(a, b):
    return lax.dot_general(a, b, (((0,), (0,)), ((), ())),
                           preferred_element_type=F32)


def _dot_c01(a, b):
    return lax.dot_general(a, b, (((0,), (1,)), ((), ())),
                           preferred_element_type=F32)


def _full_spec(shape):
    return pl.BlockSpec(shape, lambda i: tuple(0 for _ in shape))


def _tc_prep(xn, gcat, Wr, Ws, Wge, b1e, Wgn, b1n, bn=2000):
    N, dn = xn.shape
    Lp = Wr.shape[1]
    Ln = Wgn.shape[1]
    dg = gcat.shape[1]
    grid = N // bn

    def body(x_ref, g_ref, wr_ref, ws_ref, wge_ref, b1e_ref, wgn_ref, b1n_ref,
             a_ref, b_ref, ge_ref, gn_ref):
        x = x_ref[...]
        a_ref[...] = _dot(x, wr_ref[...])
        b_ref[...] = _dot(x, ws_ref[...])
        ge_ref[...] = _dot(g_ref[...], wge_ref[...]) + b1e_ref[...]
        gn_ref[...] = _dot(g_ref[...], wgn_ref[...]) + b1n_ref[...]

    return pl.pallas_call(
        body,
        grid=(grid,),
        in_specs=[
            pl.BlockSpec((bn, dn), lambda i: (i, 0)),
            _full_spec((1, dg)),
            _full_spec((dn, Lp)),
            _full_spec((dn, Lp)),
            _full_spec((dg, Lp)),
            _full_spec((1, Lp)),
            _full_spec((dg, Ln)),
            _full_spec((1, Ln)),
        ],
        out_specs=[
            pl.BlockSpec((bn, Lp), lambda i: (i, 0)),
            pl.BlockSpec((bn, Lp), lambda i: (i, 0)),
            _full_spec((1, Lp)),
            _full_spec((1, Ln)),
        ],
        out_shape=[
            jax.ShapeDtypeStruct((N, Lp), F32),
            jax.ShapeDtypeStruct((N, Lp), F32),
            jax.ShapeDtypeStruct((1, Lp), F32),
            jax.ShapeDtypeStruct((1, Ln), F32),
        ],
    )(xn, gcat, Wr, Ws, Wge, b1e, Wgn, b1n)


def _sc_gather(A, B, rowi, coli):
    N, Lp = A.shape
    M = rowi.shape[0]
    nch = M // CHUNK
    nw = NC * NS
    kmax = -(-nch // nw)
    mesh = plsc.VectorSubcoreMesh(core_axis_name="c", subcore_axis_name="s",
                                  num_cores=NC, num_subcores=NS)

    @functools.partial(
        pl.kernel,
        out_type=jax.ShapeDtypeStruct((2, M, Lp), F32),
        mesh=mesh,
        scratch_types=(
            [pltpu.VMEM((CHUNK,), jnp.int32)] * 4
            + [pltpu.VMEM((CHUNK, Lp), F32)] * 4
            + [pltpu.SemaphoreType.DMA] * 2
        ),
    )
    def k(a_hbm, b_hbm, row_hbm, col_hbm, out_hbm, rowv0, colv0, rowv1,
          colv1, bufa0, bufb0, bufa1, bufb1, sem0, sem1):
        w = lax.axis_index("s") * NC + lax.axis_index("c")
        rowv = (rowv0, rowv1)
        colv = (colv0, colv1)
        bufa = (bufa0, bufa1)
        bufb = (bufb0, bufb1)
        sem = (sem0, sem1)

        def issue(g, b):
            ch = g * nw + w

            @pl.when(ch < nch)
            def _():
                base = ch * CHUNK
                pltpu.sync_copy(row_hbm.at[pl.ds(base, CHUNK)], rowv[b])
                pltpu.sync_copy(col_hbm.at[pl.ds(base, CHUNK)], colv[b])
                pltpu.async_copy(a_hbm.at[colv[b]], bufa[b], sem[b])
                pltpu.async_copy(b_hbm.at[rowv[b]], bufb[b], sem[b])

        def drain_write(g, b):
            ch = g * nw + w

            @pl.when(ch < nch)
            def _():
                pltpu.make_async_copy(a_hbm.at[colv[b]], bufa[b],
                                      sem[b]).wait()
                pltpu.make_async_copy(b_hbm.at[rowv[b]], bufb[b],
                                      sem[b]).wait()
                base = ch * CHUNK
                pltpu.sync_copy(bufa[b], out_hbm.at[0, pl.ds(base, CHUNK)])
                pltpu.sync_copy(bufb[b], out_hbm.at[1, pl.ds(base, CHUNK)])

        issue(0, 0)

        @pl.loop(0, kmax // 2)
        def _pairs(kk):
            g0 = kk * 2
            issue(g0 + 1, 1)
            drain_write(g0, 0)
            issue(g0 + 2, 0)
            drain_write(g0 + 1, 1)

    return k(A, B, rowi, coli)


def _sc_scatter_wide(eT, rowi, coli, zeros, n_nodes):
    D, M = eT.shape
    FPW = D // (NC * NS)
    CH = 2000
    nch = M // CH
    eT_flat = eT.reshape(-1)
    mesh = plsc.VectorSubcoreMesh(core_axis_name="c", subcore_axis_name="s",
                                  num_cores=NC, num_subcores=NS)

    @functools.partial(
        pl.kernel,
        out_type=[jax.ShapeDtypeStruct((D * n_nodes,), F32)] * 2,
        mesh=mesh,
        compiler_params=pltpu.CompilerParams(needs_layout_passes=False),
        scratch_types=(
            [pltpu.VMEM((CH,), jnp.int32)] * 2
            + [pltpu.VMEM((CH,), F32)] * FPW
            + [pltpu.VMEM((n_nodes,), F32)] * (2 * FPW)
        ),
    )
    def k(e_hbm, row_hbm, col_hbm, z_hbm, s_hbm, r_hbm, rowv, colv, *bufs):
        dbuf = bufs[:FPW]
        accs = bufs[FPW:2 * FPW]
        accr = bufs[2 * FPW:3 * FPW]
        w = lax.axis_index("s") * NC + lax.axis_index("c")
        f0 = w * FPW
        for j in range(FPW):
            pltpu.sync_copy(z_hbm.at[0], accs[j])
            pltpu.sync_copy(z_hbm.at[0], accr[j])

        @pl.loop(0, nch)
        def _chunks(ch):
            base = ch * CH
            pltpu.sync_copy(row_hbm.at[pl.ds(base, CH)], rowv)
            pltpu.sync_copy(col_hbm.at[pl.ds(base, CH)], colv)
            for j in range(FPW):
                pltpu.sync_copy(e_hbm.at[pl.ds((f0 + j) * M + base, CH)],
                                dbuf[j])
            for v in range(CH // 16):
                sl = pl.ds(v * 16, 16)
                r16 = rowv[sl]
                c16 = colv[sl]
                for j in range(FPW):
                    x16 = dbuf[j][sl]
                    plsc.addupdate_scatter(accs[j], [r16], x16)
                    plsc.addupdate_scatter(accr[j], [c16], x16)

        for j in range(FPW):
            pltpu.sync_copy(accs[j],
                            s_hbm.at[pl.ds((f0 + j) * n_nodes, n_nodes)])
            pltpu.sync_copy(accr[j],
                            r_hbm.at[pl.ds((f0 + j) * n_nodes, n_nodes)])

    s_flat, r_flat = k(eT_flat, rowi, coli, zeros[:FPW])
    return s_flat.reshape(D, n_nodes), r_flat.reshape(D, n_nodes)


def _sc_scatter_narrow(eT, rowi, coli, zeros, n_nodes):
    D, M = eT.shape
    CH = 2000
    nch = M // CH
    eT_flat = eT.reshape(-1)
    mesh = plsc.VectorSubcoreMesh(core_axis_name="c", subcore_axis_name="s",
                                  num_cores=NC, num_subcores=NS)

    @functools.partial(
        pl.kernel,
        out_type=[jax.ShapeDtypeStruct((D * n_nodes,), F32)] * 2,
        mesh=mesh,
        compiler_params=pltpu.CompilerParams(needs_layout_passes=False),
        scratch_types=[
            pltpu.VMEM((CH,), jnp.int32),
            pltpu.VMEM((CH,), F32),
            pltpu.VMEM((n_nodes,), F32),
        ],
    )
    def k(e_hbm, row_hbm, col_hbm, z_hbm, s_hbm, r_hbm, idxv, dbuf, acc):
        c = lax.axis_index("c")
        s = lax.axis_index("s")
        pltpu.sync_copy(z_hbm.at[0], acc)

        @pl.loop(0, nch)
        def _chunks(ch):
            base = ch * CH

            @pl.when(c == 0)
            def _():
                pltpu.sync_copy(row_hbm.at[pl.ds(base, CH)], idxv)

            @pl.when(c == 1)
            def _():
                pltpu.sync_copy(col_hbm.at[pl.ds(base, CH)], idxv)

            pltpu.sync_copy(e_hbm.at[pl.ds(s * M + base, CH)], dbuf)
            for v in range(CH // 16):
                sl = pl.ds(v * 16, 16)
                plsc.addupdate_scatter(acc, [idxv[sl]], dbuf[sl])

        @pl.when(c == 0)
        def _():
            pltpu.sync_copy(acc, s_hbm.at[pl.ds(s * n_nodes, n_nodes)])

        @pl.when(c == 1)
        def _():
            pltpu.sync_copy(acc, r_hbm.at[pl.ds(s * n_nodes, n_nodes)])

    s_flat, r_flat = k(eT_flat, rowi, coli, zeros[:1])
    return s_flat.reshape(D, n_nodes), r_flat.reshape(D, n_nodes)


def _tc_edge(G, e_pieces, W1e_pieces, gvec, W2p, b2T, gammaT, betaT, act,
             norm, narrow, bm=3200):
    _, M, Lp = G.shape
    dout = W2p.shape[1]
    grid = M // bm
    ne = len(e_pieces)

    def body(*refs):
        g_ref = refs[0]
        e_refs = refs[1:1 + ne]
        w1_refs = refs[1 + ne:1 + 2 * ne]
        gvec_ref, w2_ref, b2t_ref = refs[1 + 2 * ne:4 + 2 * ne]
        if narrow:
            outT_ref, outN_ref, esum_ref = refs[-3:]
        else:
            outT_ref, esum_ref = refs[-2:]
        g = g_ref[...]
        h = g[0] + g[1] + gvec_ref[...]
        for e_ref, w1_ref in zip(e_refs, w1_refs):
            e = e_ref[...]
            if e.shape[0] == bm:
                h = h + _dot(e, w1_ref[...])
            else:
                h = h + _dot_c00(e, w1_ref[...])
        h = _relu(h)
        yT = _dot_c01(w2_ref[...], h) + b2t_ref[...]
        if act:
            yT = _relu(yT)
        if norm:
            yT = _layernorm(yT, refs[4 + 2 * ne][...], refs[5 + 2 * ne][...],
                            axis=0)
        outT_ref[...] = yT
        if narrow:
            outN_ref[...] = yT.T
        i = pl.program_id(0)

        @pl.when(i == 0)
        def _():
            esum_ref[...] = jnp.zeros_like(esum_ref)

        esum_ref[...] += jnp.sum(yT, axis=1, keepdims=True)

    in_specs = [pl.BlockSpec((2, bm, Lp), lambda i: (0, i, 0))]
    args = [G]
    for e in e_pieces:
        if e.shape[0] == M:
            in_specs.append(pl.BlockSpec((bm, e.shape[1]), lambda i: (i, 0)))
        else:
            in_specs.append(pl.BlockSpec((e.shape[0], bm), lambda i: (0, i)))
        args.append(e)
    for wp in W1e_pieces:
        in_specs.append(_full_spec(wp.shape))
        args.append(wp)
    for a in (gvec, W2p, b2T):
        in_specs.append(_full_spec(a.shape))
        args.append(a)
    if norm:
        for a in (gammaT, betaT):
            in_specs.append(_full_spec(a.shape))
            args.append(a)

    out_specs = [pl.BlockSpec((dout, bm), lambda i: (0, i))]
    out_shape = [jax.ShapeDtypeStruct((dout, M), F32)]
    if narrow:
        out_specs.append(pl.BlockSpec((bm, dout), lambda i: (i, 0)))
        out_shape.append(jax.ShapeDtypeStruct((M, dout), F32))
    out_specs.append(_full_spec((dout, 1)))
    out_shape.append(jax.ShapeDtypeStruct((dout, 1), F32))

    return pl.pallas_call(
        body,
        grid=(grid,),
        in_specs=in_specs,
        out_specs=out_specs,
        out_shape=out_shape,
    )(*args)


def _tc_node(xn, sendT, recvT, gvec_n, Wx, Wrecv, Wsend, W2, b2, gamma,
             beta, act, norm):
    N, dn = xn.shape
    dout = W2.shape[1]

    def body(*refs):
        (x_ref, st_ref, rt_ref, gvec_ref, wx_ref, wr_ref, ws_ref, w2_ref,
         b2_ref) = refs[:9]
        out_ref, nsum_ref = refs[-2:]
        h = (_dot(x_ref[...], wx_ref[...]) + _dot_c00(rt_ref[...], wr_ref[...])
             + _dot_c00(st_ref[...], ws_ref[...]) + gvec_ref[...])
        h = _relu(h)
        y = _dot(h, w2_ref[...]) + b2_ref[...]
        if act:
            y = _relu(y)
        if norm:
            y = _layernorm(y, refs[9][...], refs[10][...], axis=-1)
        out_ref[...] = y
        nsum_ref[...] = jnp.sum(y, axis=0, keepdims=True)

    args = [xn, sendT, recvT, gvec_n, Wx, Wrecv, Wsend, W2, b2]
    if norm:
        args += [gamma, beta]

    return pl.pallas_call(
        body,
        out_shape=[
            jax.ShapeDtypeStruct((N, dout), F32),
            jax.ShapeDtypeStruct((1, dout), F32),
        ],
    )(*args)


def _tc_global(nsum, esum, gcat, Wn, We, Wg, b1, W2, b2, gamma, beta, act,
               norm, n_nodes, n_edges):
    dout = W2.shape[1]

    def body(*refs):
        (ns_ref, es_ref, g_ref, wn_ref, we_ref, wg_ref, b1_ref, w2_ref,
         b2_ref) = refs[:9]
        out_ref = refs[-1]
        n2g = ns_ref[...] * (1.0 / n_nodes)
        e2g = es_ref[...] * (1.0 / n_edges)
        h = (_dot(n2g, wn_ref[...]) + _dot(e2g, we_ref[...])
             + _dot(g_ref[...], wg_ref[...]) + b1_ref[...])
        h = _relu(h)
        y = _dot(h, w2_ref[...]) + b2_ref[...]
        if act:
            y = _relu(y)
        if norm:
            y = _layernorm(y, refs[9][...], refs[10][...], axis=-1)
        out_ref[...] = y

    args = [nsum, esum, gcat, Wn, We, Wg, b1, W2, b2]
    if norm:
        args += [gamma, beta]
    return pl.pallas_call(
        body,
        out_shape=jax.ShapeDtypeStruct((1, dout), F32),
    )(*args)


def _pad_cols(w, Lp):
    pad = Lp - w.shape[-1]
    return jnp.pad(w, [(0, 0)] * (w.ndim - 1) + [(0, pad)]) if pad else w


def _run_block(p, e_pieces, n_pieces, g_pieces, rowi, coli, zeros, act,
               norm, narrow):
    xn = n_pieces[0] if len(n_pieces) == 1 else jnp.concatenate(n_pieces, 1)
    gcat = g_pieces[0] if len(g_pieces) == 1 else jnp.concatenate(g_pieces, 1)
    N, dn = xn.shape
    M = rowi.shape[0]

    def piece_dim(e):
        return e.shape[1] if e.shape[0] == M else e.shape[0]

    de = sum(piece_dim(e) for e in e_pieces)
    dg = gcat.shape[1]

    pe, pn, pg = p["edge"], p["node"], p["global"]
    L = pe["W1"].shape[1]
    Lp = ((L + 127) // 128) * 128

    W1 = pe["W1"]
    Wr = _pad_cols(W1[:dn], Lp)
    Ws = _pad_cols(W1[dn:2 * dn], Lp)
    We_full = W1[2 * dn:2 * dn + de]
    Wg_e = _pad_cols(W1[2 * dn + de:], Lp)
    b1e = _pad_cols(pe["b1"][None, :], Lp)
    W2e = jnp.pad(pe["W2"], ((0, Lp - L), (0, 0))) if Lp != L else pe["W2"]

    W1n = pn["W1"]
    dout_e = pe["W2"].shape[1]
    Wx = W1n[:dn]
    Wg_n = W1n[dn:dn + dg]
    Wrecv = W1n[dn + dg:dn + dg + dout_e]
    Wsend = W1n[dn + dg + dout_e:]
    b1n = pn["b1"][None, :]

    A, B, gvec_e, gvec_n = _tc_prep(xn, gcat, Wr, Ws, Wg_e, b1e, Wg_n, b1n)
    G = _sc_gather(A, B, rowi, coli)

    W1e_pieces = []
    off = 0
    for e in e_pieces:
        d = piece_dim(e)
        W1e_pieces.append(_pad_cols(We_full[off:off + d], Lp))
        off += d

    gammaT = pe["gamma"][:, None] if norm else None
    betaT = pe["beta"][:, None] if norm else None
    eout = _tc_edge(G, e_pieces, W1e_pieces, gvec_e, W2e,
                    pe["b2"][:, None], gammaT, betaT, act, norm, narrow)
    if narrow:
        eT, e_nat, esumT = eout
    else:
        eT, esumT = eout
        e_nat = None
    esum = esumT.reshape(1, dout_e)

    if narrow:
        sendT, recvT = _sc_scatter_narrow(eT, rowi, coli, zeros, N)
    else:
        sendT, recvT = _sc_scatter_wide(eT, rowi, coli, zeros, N)

    gamma_n = pn["gamma"][None, :] if norm else None
    beta_n = pn["beta"][None, :] if norm else None
    n_new, nsum = _tc_node(xn, sendT, recvT, gvec_n, Wx, Wrecv, Wsend,
                           pn["W2"], pn["b2"][None, :], gamma_n, beta_n,
                           act, norm)

    W1g = pg["W1"]
    dout_n = pn["W2"].shape[1]
    Wgn2g = W1g[:dout_n]
    Wge2g = W1g[dout_n:dout_n + dout_e]
    Wgg = W1g[dout_n + dout_e:]
    gamma_g = pg["gamma"][None, :] if norm else None
    beta_g = pg["beta"][None, :] if norm else None
    g_new = _tc_global(nsum, esum, gcat, Wgn2g, Wge2g, Wgg,
                       pg["b1"][None, :], pg["W2"], pg["b2"][None, :],
                       gamma_g, beta_g, act, norm, N, M)
    e_ret = e_nat if narrow else eT
    return e_ret, n_new, g_new


def kernel(edge_attr, edge_index, x, u, batch, params):
    rowi = edge_index[0]
    coli = edge_index[1]
    n_nodes = x.shape[0]
    zeros = jnp.zeros((4, n_nodes), F32)
    eT, n, g = _run_block(params["encoder"], [edge_attr], [x], [u], rowi,
                          coli, zeros, True, True, False)
    eT0, n0, g0 = eT, n, g
    out = None
    for _ in range(3):
        eTp, np_, gp = _run_block(params["processor"], [eT0, eT], [n0, n],
                                  [g0, g], rowi, coli, zeros, True, True,
                                  False)
        eTd, nd, gd = _run_block(params["decoder"], [eTp], [np_], [gp],
                                 rowi, coli, zeros, True, True, False)
        out = _run_block(params["output"], [eTd], [nd], [gd], rowi, coli,
                         zeros, False, False, True)
        eT, n, g = eTp, np_, gp
    return out
